# Initial kernel scaffold; baseline (speedup 1.0000x reference)
import jax
import jax.numpy as jnp
from jax.experimental import pallas as pl


def kernel(x, W, b):
    raise NotImplementedError("write your pallas kernel here")


# speedup vs baseline: 114.0477x; 114.0477x over previous
import jax
import jax.numpy as jnp
from jax.experimental import pallas as pl

SEQ = 1024
DIM = 128
KEEP = int(0.3 * SEQ * SEQ)

_HI = jax.lax.Precision.HIGHEST
_MININT = -(2**31)


def _gcn_kernel(x_ref, w_ref, b_ref, out_ref):
    xs = x_ref[...]

    ew = jax.lax.dot_general(
        xs, xs, (((1,), (1,)), ((), ())),
        preferred_element_type=jnp.float32)
    ii = jax.lax.broadcasted_iota(jnp.int32, (SEQ, SEQ), 0)
    jj = jax.lax.broadcasted_iota(jnp.int32, (SEQ, SEQ), 1)
    ew = ew - jnp.where(ii == jj, 1.0, 0.0).astype(jnp.float32)

    ibits = jax.lax.bitcast_convert_type(ew, jnp.int32)
    key = jnp.where(ibits >= 0, ibits, ibits ^ 0x7FFFFFFF)

    def value_step(i, obits):
        cand = obits | jnp.left_shift(1, 31 - i)
        cnt = jnp.sum((key >= (cand ^ _MININT)).astype(jnp.int32))
        return jnp.where(cnt >= KEEP, cand, obits)

    obits = jax.lax.fori_loop(0, 32, value_step, jnp.int32(0))
    t_key = obits ^ _MININT

    above = key > t_key
    tie = key == t_key
    n_above = jnp.sum(above.astype(jnp.int32))
    r = KEEP - n_above

    flat = ii * SEQ + jj

    def index_step(i, f):
        cand = f | jnp.left_shift(1, 20 - i)
        cnt = jnp.sum((tie & (flat < cand)).astype(jnp.int32))
        return jnp.where(cnt <= r, cand, f)

    f_cut = jax.lax.fori_loop(0, 21, index_step, jnp.int32(0))
    keep = above | (tie & (flat < f_cut))

    a = jnp.where(keep, ew, 0.0)
    ones = jnp.full((SEQ, 1), 1.0, dtype=jnp.float32)
    deg = jax.lax.dot_general(
        a, ones, (((0,), (0,)), ((), ())),
        preferred_element_type=jnp.float32, precision=_HI) + 1.0
    dis = deg ** -0.5
    dis = jnp.where(jnp.isinf(dis), 0.0, dis)

    xt = jax.lax.dot_general(
        xs, w_ref[...], (((1,), (1,)), ((), ())),
        preferred_element_type=jnp.float32, precision=_HI)

    scaled = a * dis
    y = jax.lax.dot_general(
        scaled, xt, (((0,), (0,)), ((), ())),
        preferred_element_type=jnp.float32, precision=_HI)

    out_ref[...] = dis * y + (dis * dis) * xt + b_ref[...]


def kernel(x, W, b):
    xs = x.reshape(SEQ, DIM)
    b2 = b.reshape(1, DIM)
    out = pl.pallas_call(
        _gcn_kernel,
        out_shape=jax.ShapeDtypeStruct((SEQ, DIM), jnp.float32),
    )(xs, W, b2)
    return out[None, :, :]


# speedup vs baseline: 170.3151x; 1.4934x over previous
import jax
import jax.numpy as jnp
from jax.experimental import pallas as pl

SEQ = 1024
DIM = 128
KEEP = int(0.3 * SEQ * SEQ)

_HI = jax.lax.Precision.HIGHEST
_MININT = -(2**31)


def _gcn_kernel(x_ref, w_ref, b_ref, out_ref):
    xs = x_ref[...]

    ew = jax.lax.dot_general(
        xs, xs, (((1,), (1,)), ((), ())),
        preferred_element_type=jnp.float32)
    ii = jax.lax.broadcasted_iota(jnp.int32, (SEQ, SEQ), 0)
    jj = jax.lax.broadcasted_iota(jnp.int32, (SEQ, SEQ), 1)
    ew = ew - jnp.where(ii == jj, 1.0, 0.0).astype(jnp.float32)

    ibits = jax.lax.bitcast_convert_type(ew, jnp.int32)
    key = jnp.where(ibits >= 0, ibits, ibits ^ 0x7FFFFFFF)

    def value_step(i, obits):
        cand = obits | jnp.left_shift(1, 31 - i)
        cnt = jnp.sum((key >= (cand ^ _MININT)).astype(jnp.int32))
        return jnp.where(cnt >= KEEP, cand, obits)

    obits = jax.lax.fori_loop(0, 32, value_step, jnp.int32(0))
    t_key = obits ^ _MININT

    above = key > t_key
    tie = key == t_key
    n_above = jnp.sum(above.astype(jnp.int32))
    r_f = (KEEP - n_above).astype(jnp.float32)

    tie_bf = jnp.where(tie, 1.0, 0.0).astype(jnp.bfloat16)
    l_bf = jnp.where(jj < ii, 1.0, 0.0).astype(jnp.bfloat16)
    wc = jax.lax.dot_general(
        l_bf, tie_bf, (((1,), (0,)), ((), ())),
        preferred_element_type=jnp.float32)
    rc = wc[SEQ - 1:SEQ, :] + tie[SEQ - 1:SEQ, :].astype(jnp.float32)
    inc = rc
    s = 1
    while s < SEQ:
        inc = inc + jnp.concatenate(
            [jnp.zeros((1, s), jnp.float32), inc[:, :SEQ - s]], axis=1)
        s *= 2
    row_off = inc - rc

    keep_t = above | (tie & ((row_off + wc) < r_f))
    at = jnp.where(keep_t, ew, 0.0)

    deg = jnp.sum(at, axis=1, keepdims=True) + 1.0
    dis = deg ** -0.5
    dis = jnp.where(jnp.isinf(dis), 0.0, dis)

    xt = jax.lax.dot_general(
        xs, w_ref[...], (((1,), (1,)), ((), ())),
        preferred_element_type=jnp.float32, precision=_HI)

    sx = dis * xt
    y = jax.lax.dot_general(
        at, sx, (((1,), (0,)), ((), ())),
        preferred_element_type=jnp.float32, precision=_HI)

    out_ref[...] = dis * y + (dis * dis) * xt + b_ref[...]


def kernel(x, W, b):
    xs = x.reshape(SEQ, DIM)
    b2 = b.reshape(1, DIM)
    out = pl.pallas_call(
        _gcn_kernel,
        out_shape=jax.ShapeDtypeStruct((SEQ, DIM), jnp.float32),
    )(xs, W, b2)
    return out[None, :, :]
